# baseline (speedup 1.0000x reference)
import functools

import jax
import jax.numpy as jnp
from jax.experimental import pallas as pl
from jax.experimental.pallas import tpu as pltpu

_B, _N, _D = 8, 512, 256
_K = 8
_BB = 1


def _topk_keep(P, U):
    work = P
    kf = jnp.float32(_K)
    for t in range(_K):
        m = jnp.max(work, axis=1, keepdims=True)
        if t == 0:
            thr, need = m, jnp.full_like(m, kf)
            cum = None
        else:
            open_ = cum < kf
            thr = jnp.where(open_, m, thr)
            need = jnp.where(open_, kf - cum, need)
        if t + 1 < _K:
            eqm = work == m
            c = jnp.sum(eqm.astype(jnp.float32), axis=1, keepdims=True)
            cum = c if cum is None else cum + c
            work = jnp.where(eqm, -jnp.inf, work)
    gt = P > thr
    eqt = P == thr
    pref = jax.lax.dot_general(
        eqt.astype(jnp.float32), U,
        dimension_numbers=(((1,), (0,)), ((), ())),
        preferred_element_type=jnp.float32)
    keep = gt | (eqt & (pref <= need))
    return jnp.where(keep, P, 0.0)


def _dotT(x, y):
    return jax.lax.dot_general(
        x, y, dimension_numbers=(((1,), (1,)), ((), ())),
        preferred_element_type=jnp.float32)


def _body(a_ref, b_ref, t1_ref, t2_ref, ta_ref, tb_ref, u_ref,
          oa_ref, ob_ref, oi_ref):
    sig = jax.nn.sigmoid
    U = u_ref[...]
    t1 = t1_ref[...]
    t2 = t2_ref[...]
    ta = ta_ref[...]
    tb = tb_ref[...]

    for j in range(_BB):
        a = a_ref[j]
        b = b_ref[j]

        pa = sig(_dotT(a * t1, a * t2))
        oa_ref[j] = _topk_keep(jax.nn.relu(pa - pa.T), U)

        pb = sig(_dotT(b * t1, b * t2))
        ob_ref[j] = _topk_keep(jax.nn.relu(pb - pb.T), U)

        oi_ref[j] = _topk_keep(sig(_dotT(a * ta, b * tb)), U)


@functools.partial(jax.jit, static_argnames=())
def kernel(tensor_a, tensor_b, theta1_intra, theta2_intra,
           theta_a_inter, theta_b_inter):
    t1 = theta1_intra.reshape(1, _D)
    t2 = theta2_intra.reshape(1, _D)
    ta = theta_a_inter.reshape(1, _D)
    tb = theta_b_inter.reshape(1, _D)

    row = jax.lax.broadcasted_iota(jnp.int32, (_N, _N), 0)
    col = jax.lax.broadcasted_iota(jnp.int32, (_N, _N), 1)
    upper = (row <= col).astype(jnp.float32)

    batch_spec = pl.BlockSpec((_BB, _N, _D), lambda i: (i, 0, 0))
    theta_spec = pl.BlockSpec((1, _D), lambda i: (0, 0))
    const_spec = pl.BlockSpec((_N, _N), lambda i: (0, 0))
    out_spec = pl.BlockSpec((_BB, _N, _N), lambda i: (i, 0, 0))
    out_shape = jax.ShapeDtypeStruct((_B, _N, _N), jnp.float32)

    return pl.pallas_call(
        _body,
        grid=(_B // _BB,),
        in_specs=[batch_spec, batch_spec,
                  theta_spec, theta_spec, theta_spec, theta_spec,
                  const_spec],
        out_specs=[out_spec, out_spec, out_spec],
        out_shape=[out_shape, out_shape, out_shape],
        compiler_params=pltpu.CompilerParams(
            dimension_semantics=("parallel",)),
    )(tensor_a, tensor_b, t1, t2, ta, tb, upper)


# speedup vs baseline: 1.0196x; 1.0196x over previous
import functools

import jax
import jax.numpy as jnp
from jax.experimental import pallas as pl
from jax.experimental.pallas import tpu as pltpu

_B, _N, _D = 8, 512, 256
_K = 8
_BB = 1


def _topk_keep(P, U):
    work = P
    kf = jnp.float32(_K)
    for t in range(_K):
        m = jnp.max(work, axis=1, keepdims=True)
        if t == 0:
            thr, need = m, jnp.full_like(m, kf)
            cum = None
        else:
            open_ = cum < kf
            thr = jnp.where(open_, m, thr)
            need = jnp.where(open_, kf - cum, need)
        if t + 1 < _K:
            eqm = work == m
            c = jnp.sum(eqm.astype(jnp.float32), axis=1, keepdims=True)
            cum = c if cum is None else cum + c
            work = jnp.where(eqm, -jnp.inf, work)
    gt = P > thr
    eqt = P == thr
    pref = jax.lax.dot_general(
        eqt.astype(jnp.float32), U,
        dimension_numbers=(((1,), (0,)), ((), ())),
        preferred_element_type=jnp.float32)
    keep = gt | (eqt & (pref <= need))
    return jnp.where(keep, P, 0.0)


def _dotT(x, y):
    return jax.lax.dot_general(
        x, y, dimension_numbers=(((1,), (1,)), ((), ())),
        preferred_element_type=jnp.float32)


def _body(a_ref, b_ref, t1_ref, t2_ref, ta_ref, tb_ref,
          oa_ref, ob_ref, oi_ref):
    sig = jax.nn.sigmoid
    U = (jax.lax.broadcasted_iota(jnp.int32, (_N, _N), 0)
         <= jax.lax.broadcasted_iota(jnp.int32, (_N, _N), 1)).astype(jnp.float32)
    t1 = t1_ref[...]
    t2 = t2_ref[...]
    ta = ta_ref[...]
    tb = tb_ref[...]

    for j in range(_BB):
        a = a_ref[j]
        b = b_ref[j]

        pa = sig(_dotT(a * t1, a * t2))
        oa_ref[j] = _topk_keep(jax.nn.relu(pa - pa.T), U)

        pb = sig(_dotT(b * t1, b * t2))
        ob_ref[j] = _topk_keep(jax.nn.relu(pb - pb.T), U)

        oi_ref[j] = _topk_keep(sig(_dotT(a * ta, b * tb)), U)


@functools.partial(jax.jit, static_argnames=())
def kernel(tensor_a, tensor_b, theta1_intra, theta2_intra,
           theta_a_inter, theta_b_inter):
    t1 = theta1_intra.reshape(1, _D)
    t2 = theta2_intra.reshape(1, _D)
    ta = theta_a_inter.reshape(1, _D)
    tb = theta_b_inter.reshape(1, _D)

    batch_spec = pl.BlockSpec((_BB, _N, _D), lambda i: (i, 0, 0))
    theta_spec = pl.BlockSpec((1, _D), lambda i: (0, 0))
    out_spec = pl.BlockSpec((_BB, _N, _N), lambda i: (i, 0, 0))
    out_shape = jax.ShapeDtypeStruct((_B, _N, _N), jnp.float32)

    return pl.pallas_call(
        _body,
        grid=(_B // _BB,),
        in_specs=[batch_spec, batch_spec,
                  theta_spec, theta_spec, theta_spec, theta_spec],
        out_specs=[out_spec, out_spec, out_spec],
        out_shape=[out_shape, out_shape, out_shape],
        compiler_params=pltpu.CompilerParams(
            dimension_semantics=("parallel",)),
    )(tensor_a, tensor_b, t1, t2, ta, tb)


# speedup vs baseline: 1.0723x; 1.0517x over previous
import functools

import jax
import jax.numpy as jnp
from jax.experimental import pallas as pl
from jax.experimental.pallas import tpu as pltpu

_B, _N, _D = 8, 512, 256
_K = 8
_BB = 1


def _topk_keep_multi(Ps, U):
    n = len(Ps)
    kf = jnp.float32(_K)
    works = list(Ps)
    thrs = [None] * n
    needs = [None] * n
    cums = [None] * n
    for t in range(_K):
        ms = [jnp.max(w, axis=1, keepdims=True) for w in works]
        for i in range(n):
            if t == 0:
                thrs[i], needs[i] = ms[i], jnp.full_like(ms[i], kf)
            else:
                open_ = cums[i] < kf
                thrs[i] = jnp.where(open_, ms[i], thrs[i])
                needs[i] = jnp.where(open_, kf - cums[i], needs[i])
        if t + 1 < _K:
            eqs = [w == m for w, m in zip(works, ms)]
            for i in range(n):
                c = jnp.sum(eqs[i].astype(jnp.float32), axis=1, keepdims=True)
                cums[i] = c if cums[i] is None else cums[i] + c
            works = [jnp.where(e, -jnp.inf, w) for e, w in zip(eqs, works)]
    outs = []
    for P, thr, need in zip(Ps, thrs, needs):
        gt = P > thr
        eqt = P == thr
        pref = jax.lax.dot_general(
            eqt.astype(jnp.float32), U,
            dimension_numbers=(((1,), (0,)), ((), ())),
            preferred_element_type=jnp.float32)
        outs.append(jnp.where(gt | (eqt & (pref <= need)), P, 0.0))
    return outs


def _topk_keep(P, U):
    work = P
    kf = jnp.float32(_K)
    for t in range(_K):
        m = jnp.max(work, axis=1, keepdims=True)
        if t == 0:
            thr, need = m, jnp.full_like(m, kf)
            cum = None
        else:
            open_ = cum < kf
            thr = jnp.where(open_, m, thr)
            need = jnp.where(open_, kf - cum, need)
        if t + 1 < _K:
            eqm = work == m
            c = jnp.sum(eqm.astype(jnp.float32), axis=1, keepdims=True)
            cum = c if cum is None else cum + c
            work = jnp.where(eqm, -jnp.inf, work)
    gt = P > thr
    eqt = P == thr
    pref = jax.lax.dot_general(
        eqt.astype(jnp.float32), U,
        dimension_numbers=(((1,), (0,)), ((), ())),
        preferred_element_type=jnp.float32)
    keep = gt | (eqt & (pref <= need))
    return jnp.where(keep, P, 0.0)


def _dotT(x, y):
    return jax.lax.dot_general(
        x, y, dimension_numbers=(((1,), (1,)), ((), ())),
        preferred_element_type=jnp.float32)


def _body(a_ref, b_ref, t1_ref, t2_ref, ta_ref, tb_ref,
          oa_ref, ob_ref, oi_ref):
    sig = jax.nn.sigmoid
    U = (jax.lax.broadcasted_iota(jnp.int32, (_N, _N), 0)
         <= jax.lax.broadcasted_iota(jnp.int32, (_N, _N), 1)).astype(jnp.float32)
    t1 = t1_ref[...]
    t2 = t2_ref[...]
    ta = ta_ref[...]
    tb = tb_ref[...]

    for j in range(_BB):
        a = a_ref[j]
        b = b_ref[j]

        pa = sig(_dotT(a * t1, a * t2))
        pb = sig(_dotT(b * t1, b * t2))
        pi = sig(_dotT(a * ta, b * tb))
        oa, ob, oi = _topk_keep_multi(
            [jax.nn.relu(pa - pa.T), jax.nn.relu(pb - pb.T), pi], U)
        oa_ref[j] = oa
        ob_ref[j] = ob
        oi_ref[j] = oi


@functools.partial(jax.jit, static_argnames=())
def kernel(tensor_a, tensor_b, theta1_intra, theta2_intra,
           theta_a_inter, theta_b_inter):
    t1 = theta1_intra.reshape(1, _D)
    t2 = theta2_intra.reshape(1, _D)
    ta = theta_a_inter.reshape(1, _D)
    tb = theta_b_inter.reshape(1, _D)

    batch_spec = pl.BlockSpec((_BB, _N, _D), lambda i: (i, 0, 0))
    theta_spec = pl.BlockSpec((1, _D), lambda i: (0, 0))
    out_spec = pl.BlockSpec((_BB, _N, _N), lambda i: (i, 0, 0))
    out_shape = jax.ShapeDtypeStruct((_B, _N, _N), jnp.float32)

    return pl.pallas_call(
        _body,
        grid=(_B // _BB,),
        in_specs=[batch_spec, batch_spec,
                  theta_spec, theta_spec, theta_spec, theta_spec],
        out_specs=[out_spec, out_spec, out_spec],
        out_shape=[out_shape, out_shape, out_shape],
        compiler_params=pltpu.CompilerParams(
            dimension_semantics=("parallel",)),
    )(tensor_a, tensor_b, t1, t2, ta, tb)
